# baseline (speedup 1.0000x reference)
import functools

import jax
import jax.numpy as jnp
from jax import lax
from jax.experimental import pallas as pl
from jax.experimental.pallas import tpu as pltpu
from jax.experimental.pallas import tpu_sc as plsc

_H = 16
_S = 2048
_WPAD = 4112
_NTILES = 32
_ROWS_PER_TILE = (_H * _S) // _NTILES
_CLS = 8
_BPC = 8
_CW = _S + 128 * (_BPC - 1)


def _sc_body(table_hbm, buckets_hbm, out_hbm, bd2_v, table_v, w_v,
             buf_a, buf_b, sem_a, sem_b):
    wid = lax.axis_index("s") * 2 + lax.axis_index("c")
    h = wid // 2
    row0 = wid * _ROWS_PER_TILE
    i0 = row0 & (_S - 1)

    pltpu.sync_copy(table_hbm, table_v)
    pltpu.sync_copy(buckets_hbm.at[_S - 1], bd2_v.at[pl.ds(0, _S)])
    pltpu.sync_copy(buckets_hbm.at[0], bd2_v.at[pl.ds(_S, _S)])

    iota16 = lax.iota(jnp.int32, 16)

    def build(k, _):
        m = iota16 + k * 16
        m = jnp.minimum(m, 2 * _S - 2)
        posv = m + (m >= _S).astype(jnp.int32)
        bidx = plsc.load_gather(bd2_v, [posv])
        vals = plsc.load_gather(table_v, [bidx * _H + h])
        w_v[pl.ds(k * 16, 16)] = vals
        return 0

    lax.fori_loop(0, _WPAD // 16, build, 0, unroll=8)

    def fill(buf, p):
        q = (1151 - i0) - 16 * p
        for s in range(16):
            base = q - s

            def chunk(kk, _, s=s, base=base):
                vals = plsc.load_gather(w_v, [iota16 + (base + kk * 16)])
                buf[s, pl.ds(kk * 16, 16)] = vals
                return 0

            lax.fori_loop(0, _CW // 16, chunk, 0, unroll=8)

    def issue(buf, p, sem):
        for k in range(_BPC):
            off = 128 * (_BPC - 1 - k)
            pltpu.async_copy(
                buf.at[:, pl.ds(off, _S)],
                out_hbm.at[pl.ds(row0 + 16 * p + 128 * k, 16)],
                sem,
            )

    def drain(buf, sem):
        for _ in range(_BPC):
            pltpu.make_async_copy(
                buf.at[:, pl.ds(0, _S)], out_hbm.at[pl.ds(row0, 16)], sem
            ).wait()

    fill(buf_a, 0)
    issue(buf_a, 0, sem_a)
    fill(buf_b, 1)
    issue(buf_b, 1, sem_b)

    def pair(pp, _):
        p = 2 * pp + 2
        drain(buf_a, sem_a)
        fill(buf_a, p)
        issue(buf_a, p, sem_a)
        drain(buf_b, sem_b)
        fill(buf_b, p + 1)
        issue(buf_b, p + 1, sem_b)
        return 0

    lax.fori_loop(0, (_CLS - 2) // 2, pair, 0)

    drain(buf_a, sem_a)
    drain(buf_b, sem_b)


@jax.jit
def _sc_call(table_flat, buckets):
    call = functools.partial(
        pl.kernel,
        mesh=plsc.VectorSubcoreMesh(core_axis_name="c", subcore_axis_name="s"),
        out_type=jax.ShapeDtypeStruct((_H * _S, _S), jnp.float32),
        scratch_types=[
            pltpu.VMEM((2 * _S + 16,), jnp.int32),
            pltpu.VMEM((32 * _H,), jnp.float32),
            pltpu.VMEM((_WPAD,), jnp.float32),
            pltpu.VMEM((16, _CW), jnp.float32),
            pltpu.VMEM((16, _CW), jnp.float32),
            pltpu.SemaphoreType.DMA,
            pltpu.SemaphoreType.DMA,
        ],
        compiler_params=pltpu.CompilerParams(needs_layout_passes=False),
    )(_sc_body)
    return call(table_flat, buckets)


def kernel(table, buckets, seq_len):
    del seq_len
    out = _sc_call(jnp.reshape(table, (-1,)), buckets)
    return jnp.reshape(out, (_H, _S, _S))


# speedup vs baseline: 1.0728x; 1.0728x over previous
import functools

import jax
import jax.numpy as jnp
from jax import lax
from jax.experimental import pallas as pl
from jax.experimental.pallas import tpu as pltpu
from jax.experimental.pallas import tpu_sc as plsc

_H = 16
_S = 2048
_WPAD = 4112
_NTILES = 32
_ROWS_PER_TILE = (_H * _S) // _NTILES
_CLS = 16
_GPC = 8
_CW = _S + 128 * (_GPC - 1)


def _sc_body(table_hbm, buckets_hbm, out_hbm, bd2_v, table_v, w_v,
             buf_a, buf_b, sem_a, sem_b):
    wid = lax.axis_index("s") * 2 + lax.axis_index("c")
    h = wid // 2
    row0 = wid * _ROWS_PER_TILE
    i0 = row0 & (_S - 1)

    pltpu.sync_copy(table_hbm, table_v)
    pltpu.sync_copy(buckets_hbm.at[_S - 1], bd2_v.at[pl.ds(0, _S)])
    pltpu.sync_copy(buckets_hbm.at[0], bd2_v.at[pl.ds(_S, _S)])

    iota16 = lax.iota(jnp.int32, 16)

    def build(k, _):
        m = iota16 + k * 16
        m = jnp.minimum(m, 2 * _S - 2)
        posv = m + (m >= _S).astype(jnp.int32)
        bidx = plsc.load_gather(bd2_v, [posv])
        vals = plsc.load_gather(table_v, [bidx * _H + h])
        w_v[pl.ds(k * 16, 16)] = vals
        return 0

    lax.fori_loop(0, _WPAD // 16, build, 0, unroll=4)

    def fill(buf, t):
        q = (1151 - i0) - 8 * t
        for s in range(8):
            base = q - s

            def chunk(kk, _, s=s, base=base):
                vals = plsc.load_gather(w_v, [iota16 + (base + kk * 16)])
                buf[s, pl.ds(kk * 16, 16)] = vals
                return 0

            lax.fori_loop(0, _CW // 16, chunk, 0, unroll=8)

    def issue(buf, t, sem):
        for k in range(_GPC):
            off = 128 * (_GPC - 1 - k)
            pltpu.async_copy(
                buf.at[:, pl.ds(off, _S)],
                out_hbm.at[pl.ds(row0 + 8 * t + 128 * k, 8)],
                sem,
            )

    def drain(buf, sem):
        for _ in range(_GPC):
            pltpu.make_async_copy(
                buf.at[:, pl.ds(0, _S)], out_hbm.at[pl.ds(row0, 8)], sem
            ).wait()

    fill(buf_a, 0)
    issue(buf_a, 0, sem_a)
    fill(buf_b, 1)
    issue(buf_b, 1, sem_b)

    def pair(p, _):
        t = 2 * p + 2
        drain(buf_a, sem_a)
        fill(buf_a, t)
        issue(buf_a, t, sem_a)
        drain(buf_b, sem_b)
        fill(buf_b, t + 1)
        issue(buf_b, t + 1, sem_b)
        return 0

    lax.fori_loop(0, (_CLS - 2) // 2, pair, 0)

    drain(buf_a, sem_a)
    drain(buf_b, sem_b)


@jax.jit
def _sc_call(table_flat, buckets):
    call = functools.partial(
        pl.kernel,
        mesh=plsc.VectorSubcoreMesh(core_axis_name="c", subcore_axis_name="s"),
        out_type=jax.ShapeDtypeStruct((_H * _S, _S), jnp.float32),
        scratch_types=[
            pltpu.VMEM((2 * _S + 16,), jnp.int32),
            pltpu.VMEM((32 * _H,), jnp.float32),
            pltpu.VMEM((_WPAD,), jnp.float32),
            pltpu.VMEM((8, _CW), jnp.float32),
            pltpu.VMEM((8, _CW), jnp.float32),
            pltpu.SemaphoreType.DMA,
            pltpu.SemaphoreType.DMA,
        ],
        compiler_params=pltpu.CompilerParams(needs_layout_passes=False),
    )(_sc_body)
    return call(table_flat, buckets)


def kernel(table, buckets, seq_len):
    del seq_len
    out = _sc_call(jnp.reshape(table, (-1,)), buckets)
    return jnp.reshape(out, (_H, _S, _S))


# speedup vs baseline: 1.0849x; 1.0113x over previous
import functools

import jax
import jax.numpy as jnp
from jax import lax
from jax.experimental import pallas as pl
from jax.experimental.pallas import tpu as pltpu
from jax.experimental.pallas import tpu_sc as plsc

_H = 16
_S = 2048
_WPAD = 4112
_NTILES = 32
_ROWS_PER_TILE = (_H * _S) // _NTILES
_CLS = 16
_GPC = 8
_CW = _S + 128 * (_GPC - 1)


def _sc_body(table_hbm, buckets_hbm, out_hbm, bd2_v, table_v, w_v,
             buf_a, buf_b, sem_a, sem_b):
    wid = lax.axis_index("s") * 2 + lax.axis_index("c")
    h = wid // 2
    row0 = wid * _ROWS_PER_TILE
    i0 = row0 & (_S - 1)

    c1 = pltpu.async_copy(table_hbm, table_v, sem_a)
    c2 = pltpu.async_copy(buckets_hbm.at[_S - 1], bd2_v.at[pl.ds(0, _S)], sem_a)
    c3 = pltpu.async_copy(buckets_hbm.at[0], bd2_v.at[pl.ds(_S, _S)], sem_a)
    c1.wait()
    c2.wait()
    c3.wait()

    iota16 = lax.iota(jnp.int32, 16)

    def build(k, _):
        m = iota16 + k * 16
        m = jnp.minimum(m, 2 * _S - 2)
        posv = m + (m >= _S).astype(jnp.int32)
        bidx = plsc.load_gather(bd2_v, [posv])
        vals = plsc.load_gather(table_v, [bidx * _H + h])
        w_v[pl.ds(k * 16, 16)] = vals
        return 0

    lax.fori_loop(0, _WPAD // 16, build, 0, unroll=4)

    def fill(buf, t):
        q = (1151 - i0) - 8 * t
        for s in range(8):
            base = q - s

            def chunk(kk, _, s=s, base=base):
                vals = plsc.load_gather(w_v, [iota16 + (base + kk * 16)])
                buf[s, pl.ds(kk * 16, 16)] = vals
                return 0

            lax.fori_loop(0, _CW // 16, chunk, 0, unroll=8)

    def issue(buf, t, sem):
        for k in range(_GPC):
            off = 128 * (_GPC - 1 - k)
            pltpu.async_copy(
                buf.at[:, pl.ds(off, _S)],
                out_hbm.at[pl.ds(row0 + 8 * t + 128 * k, 8)],
                sem,
            )

    def drain(buf, sem):
        for _ in range(_GPC):
            pltpu.make_async_copy(
                buf.at[:, pl.ds(0, _S)], out_hbm.at[pl.ds(row0, 8)], sem
            ).wait()

    fill(buf_a, 0)
    issue(buf_a, 0, sem_a)
    fill(buf_b, 1)
    issue(buf_b, 1, sem_b)

    def pair(p, _):
        t = 2 * p + 2
        drain(buf_a, sem_a)
        fill(buf_a, t)
        issue(buf_a, t, sem_a)
        drain(buf_b, sem_b)
        fill(buf_b, t + 1)
        issue(buf_b, t + 1, sem_b)
        return 0

    lax.fori_loop(0, (_CLS - 2) // 2, pair, 0)

    drain(buf_a, sem_a)
    drain(buf_b, sem_b)


@jax.jit
def _sc_call(table_flat, buckets):
    call = functools.partial(
        pl.kernel,
        mesh=plsc.VectorSubcoreMesh(core_axis_name="c", subcore_axis_name="s"),
        out_type=jax.ShapeDtypeStruct((_H * _S, _S), jnp.float32),
        scratch_types=[
            pltpu.VMEM((2 * _S + 16,), jnp.int32),
            pltpu.VMEM((32 * _H,), jnp.float32),
            pltpu.VMEM((_WPAD,), jnp.float32),
            pltpu.VMEM((8, _CW), jnp.float32),
            pltpu.VMEM((8, _CW), jnp.float32),
            pltpu.SemaphoreType.DMA,
            pltpu.SemaphoreType.DMA,
        ],
        compiler_params=pltpu.CompilerParams(needs_layout_passes=False),
    )(_sc_body)
    return call(table_flat, buckets)


def kernel(table, buckets, seq_len):
    del seq_len
    out = _sc_call(jnp.reshape(table, (-1,)), buckets)
    return jnp.reshape(out, (_H, _S, _S))
